# baseline (speedup 1.0000x reference)
import functools
import jax
import jax.numpy as jnp
from jax import lax
from jax.experimental import pallas as pl
from jax.experimental.pallas import tpu as pltpu
from jax.experimental.pallas import tpu_sc as plsc

_H = 64
_W = 64
_A = 9
_NW = 16
_NT = _H * _W * _A // 128
_TPW = _NT // _NW
_LN2 = 0.6931471805599453

_mesh = plsc.VectorSubcoreMesh(core_axis_name="c", subcore_axis_name="s", num_cores=1)


@functools.partial(
    pl.kernel,
    out_type=(
        jax.ShapeDtypeStruct((_NT, 4, 128), jnp.float32),
        jax.ShapeDtypeStruct((_NT, 4, 128), jnp.float32),
    ),
    mesh=_mesh,
    scratch_types=[
        pltpu.VMEM((_TPW, 4, 128), jnp.float32),
        pltpu.VMEM((_TPW, 4, 128), jnp.float32),
    ],
)
def _sc_gen(o1_hbm, o2_hbm, buf1, buf2):
    wid = lax.axis_index("s") + lax.axis_index("c")
    lane = lax.iota(jnp.int32, 16)
    widv = jnp.broadcast_to(wid, (16,))
    posb = widv * (128 * _TPW // 9)

    for ii in range(_TPW):
        def _body(vv, _, ii=ii):
            m = jnp.broadcast_to(vv * 16 + ii * 128, (16,)) + lane
            a = m % 9
            q = (m * 7282) >> 16
            pos = posb + q
            w = pos & 63
            h = pos >> 6
            cx = w.astype(jnp.float32) * 8.0 + 4.0
            cy = h.astype(jnp.float32) * 8.0 + 4.0
            s = (a % 3).astype(jnp.float32)
            t = ((a * 11) >> 5).astype(jnp.float32)
            e1 = s * (1.0 / 3.0)
            e2 = (t - 1.0) * 0.5
            bw = 32.0 * jnp.exp(_LN2 * (e1 + e2))
            bh = 32.0 * jnp.exp(_LN2 * (e1 - e2))
            ds = pl.ds(vv * 16, 16)
            buf1[ii, 0, ds] = cx
            buf1[ii, 1, ds] = cy
            buf1[ii, 2, ds] = bw
            buf1[ii, 3, ds] = bh
            buf2[ii, 0, ds] = cx - bw * 0.5
            buf2[ii, 1, ds] = cy - bh * 0.5
            buf2[ii, 2, ds] = cx + bw * 0.5
            buf2[ii, 3, ds] = cy + bh * 0.5
            return 0

        lax.fori_loop(0, 8, _body, 0)

    tile0 = wid * _TPW
    pltpu.sync_copy(buf1, o1_hbm.at[pl.ds(tile0, _TPW)])
    pltpu.sync_copy(buf2, o2_hbm.at[pl.ds(tile0, _TPW)])


def kernel(features):
    del features
    o1, o2 = _sc_gen()
    a1 = o1.transpose(0, 2, 1).reshape(_H * _W * _A, 4)
    a2 = o2.transpose(0, 2, 1).reshape(_H * _W * _A, 4)
    return a1, a2


# speedup vs baseline: 11.0626x; 11.0626x over previous
import jax
import jax.numpy as jnp
from jax import lax
from jax.experimental import pallas as pl

_H = 64
_W = 64
_A = 9
_NT = _H * _W * _A // 128
_ROWS = _NT * 4


_GRID = 2
_BLK = _ROWS // _GRID


def _gen_body(o1_ref, o2_ref):
    i = pl.program_id(0)
    rp = lax.broadcasted_iota(jnp.int32, (_BLK, 128), 0) + i * _BLK
    l = lax.broadcasted_iota(jnp.int32, (_BLK, 128), 1)
    n = (rp >> 2) * 128 + l
    q = lax.shift_right_logical(n * 58255, 19)
    a = n - q * 9
    t = lax.shift_right_logical(a * 11, 5)
    s = a - t * 3
    c_odd = (rp & 1) == 1
    c_low = (rp & 2) == 0
    uq = jnp.where(c_odd, lax.shift_right_logical(q, 6), q & 63)
    u = uq.astype(jnp.float32) * 8.0 + 4.0
    e1 = s.astype(jnp.float32) * (1.0 / 3.0)
    e2 = t.astype(jnp.float32) * 0.5
    v = jnp.exp2(e1 + jnp.where(c_odd, 5.5 - e2, 4.5 + e2))
    o1_ref[...] = jnp.where(c_low, u, v)
    hv = v * jnp.where(c_low, -0.5, 0.5)
    o2_ref[...] = u + hv


def kernel(features):
    del features
    o1, o2 = pl.pallas_call(
        _gen_body,
        grid=(_GRID,),
        out_specs=(
            pl.BlockSpec((_BLK, 128), lambda i: (i, 0)),
            pl.BlockSpec((_BLK, 128), lambda i: (i, 0)),
        ),
        out_shape=(
            jax.ShapeDtypeStruct((_ROWS, 128), jnp.float32),
            jax.ShapeDtypeStruct((_ROWS, 128), jnp.float32),
        ),
    )()
    a1 = o1.reshape(_NT, 4, 128).transpose(0, 2, 1).reshape(_H * _W * _A, 4)
    a2 = o2.reshape(_NT, 4, 128).transpose(0, 2, 1).reshape(_H * _W * _A, 4)
    return a1, a2
